# Initial kernel scaffold; baseline (speedup 1.0000x reference)
import jax
import jax.numpy as jnp
from jax.experimental import pallas as pl


def kernel(user_input, pos_item_input, neg_item_input, user_table, item_table):
    raise NotImplementedError("write your pallas kernel here")


# speedup vs baseline: 1.7723x; 1.7723x over previous
import functools

import jax
import jax.numpy as jnp
from jax import lax
from jax.experimental import pallas as pl
from jax.experimental.pallas import tpu as pltpu
from jax.experimental.pallas import tpu_sc as plsc

_BATCH = 16384
_D = 128
_LANES = 16
_NC = 2
_NS = 16
_NW = _NC * _NS
_BPW = _BATCH // _NW
_C = 128
_NCHUNK = _BPW // _C
_MARGIN = 1.0
_EPS2 = 1e-24


def _rsqrt(x):
    i = lax.bitcast_convert_type(x, jnp.int32)
    i = 0x5F3759DF - lax.shift_right_logical(i, 1)
    y = lax.bitcast_convert_type(i, jnp.float32)
    for _ in range(3):
        y = y * (1.5 - 0.5 * x * y * y)
    return y


def _make_sc_kernel():
    mesh = plsc.VectorSubcoreMesh(core_axis_name="c", subcore_axis_name="s")

    @functools.partial(
        pl.kernel,
        mesh=mesh,
        compiler_params=pltpu.CompilerParams(needs_layout_passes=False),
        out_type=jax.ShapeDtypeStruct((_NW, _LANES), jnp.float32),
        scratch_types=[
            pltpu.VMEM((_C,), jnp.int32),
            pltpu.VMEM((_C,), jnp.int32),
            pltpu.VMEM((_C,), jnp.int32),
            pltpu.VMEM((_C, _D), jnp.float32),
            pltpu.VMEM((_C, _D), jnp.float32),
            pltpu.VMEM((_C, _D), jnp.float32),
            pltpu.VMEM((_LANES,), jnp.float32),
            pltpu.SemaphoreType.DMA,
        ],
    )
    def sc_loss(uid_h, pid_h, nid_h, utab_h, itab_h, out_h,
                iu, ip, iv, bu, bp, bn, ov, sem):
        wid = lax.axis_index("s") * _NC + lax.axis_index("c")
        lane = lax.iota(jnp.int32, _LANES)

        def chunk_body(c, acc):
            base = wid * _BPW + c * _C
            pltpu.sync_copy(uid_h.at[pl.ds(base, _C)], iu)
            pltpu.sync_copy(pid_h.at[pl.ds(base, _C)], ip)
            pltpu.sync_copy(nid_h.at[pl.ds(base, _C)], iv)
            cp_u = pltpu.async_copy(utab_h.at[iu], bu, sem)
            cp_p = pltpu.async_copy(itab_h.at[ip], bp, sem)
            cp_n = pltpu.async_copy(itab_h.at[iv], bn, sem)
            cp_u.wait()
            cp_p.wait()
            cp_n.wait()

            def row_body(r, acc_in):
                zero = jnp.zeros((_LANES,), jnp.float32)
                uu = pp = nn = up = un = zero
                for j in range(_D // _LANES):
                    sl = pl.ds(j * _LANES, _LANES)
                    u = bu[r, sl]
                    p = bp[r, sl]
                    n = bn[r, sl]
                    uu = uu + u * u
                    pp = pp + p * p
                    nn = nn + n * n
                    up = up + u * p
                    un = un + u * n
                suu = jnp.maximum(jnp.sum(uu), _EPS2)
                spp = jnp.maximum(jnp.sum(pp), _EPS2)
                snn = jnp.maximum(jnp.sum(nn), _EPS2)
                sim_p = jnp.sum(up) * _rsqrt(suu * spp)
                sim_n = jnp.sum(un) * _rsqrt(suu * snn)
                hinge = jnp.maximum(0.0, _MARGIN - sim_p + sim_n)
                return acc_in + hinge

            return lax.fori_loop(0, _C, row_body, acc)

        total = lax.fori_loop(0, _NCHUNK, chunk_body, jnp.float32(0.0))
        ov[...] = jnp.where(lane == 0, total, 0.0)
        pltpu.sync_copy(ov, out_h.at[wid])

    return sc_loss


_sc_loss_kernel = _make_sc_kernel()


def kernel(user_input, pos_item_input, neg_item_input, user_table, item_table):
    uid = user_input.reshape(-1).astype(jnp.int32)
    pid = pos_item_input.reshape(-1).astype(jnp.int32)
    nid = neg_item_input.reshape(-1).astype(jnp.int32)
    partials = _sc_loss_kernel(uid, pid, nid, user_table, item_table)
    return jnp.sum(partials) * (1.0 / _BATCH)
